# baseline (speedup 1.0000x reference)
import functools

import jax
import jax.numpy as jnp
from jax import lax
from jax.experimental import pallas as pl
from jax.experimental.pallas import tpu as pltpu
from jax.experimental.pallas import tpu_sc as plsc

NC = 2
NS = 16
L = 16
NW = NC * NS
B = 16384
BPW = B // NW
CHUNK = 128
NCHUNK = BPW // CHUNK
ROWS = 1000000
GRP = 8
WROWS = ROWS // GRP


def kernel(user, item, users_emb, items_emb, W, b):
    uw = users_emb.reshape(WROWS, GRP * L)
    iw = items_emb.reshape(WROWS, GRP * L)
    wmat = jnp.broadcast_to(W.reshape(L, 1), (L, L))
    bv = jnp.broadcast_to(b, (L,))

    mesh = plsc.VectorSubcoreMesh(core_axis_name="c", subcore_axis_name="s")
    cp = pltpu.CompilerParams(needs_layout_passes=False)

    @functools.partial(
        pl.kernel,
        out_type=jax.ShapeDtypeStruct((B,), jnp.float32),
        mesh=mesh,
        compiler_params=cp,
        scratch_types=[
            pltpu.VMEM((BPW,), jnp.int32),
            pltpu.VMEM((BPW,), jnp.int32),
            pltpu.VMEM((BPW,), jnp.int32),
            pltpu.VMEM((BPW,), jnp.int32),
            pltpu.VMEM((CHUNK, GRP * L), jnp.float32),
            pltpu.VMEM((CHUNK, GRP * L), jnp.float32),
            pltpu.VMEM((CHUNK, GRP * L), jnp.float32),
            pltpu.VMEM((CHUNK, GRP * L), jnp.float32),
            pltpu.VMEM((L, L), jnp.float32),
            pltpu.VMEM((L,), jnp.float32),
            pltpu.VMEM((BPW,), jnp.float32),
            pltpu.SemaphoreType.DMA,
            pltpu.SemaphoreType.DMA,
        ],
    )
    def sc_kernel(user_hbm, item_hbm, uemb_hbm, iemb_hbm, wmat_hbm, bv_hbm,
                  out_hbm, idxu_v, idxi_v, iwu_v, iwi_v,
                  gu0, gu1, gi0, gi1, wmat_v, bv_v, out_v, sem_idx, sem_g):
        wid = lax.axis_index("s") * NC + lax.axis_index("c")
        base = wid * BPW

        pltpu.sync_copy(wmat_hbm, wmat_v)
        pltpu.sync_copy(bv_hbm, bv_v)

        cu = pltpu.async_copy(user_hbm.at[pl.ds(base, BPW)], idxu_v, sem_idx)
        ci = pltpu.async_copy(item_hbm.at[pl.ds(base, BPW)], idxi_v, sem_idx)
        cu.wait()
        ci.wait()

        @pl.loop(0, BPW // L)
        def _(g):
            s = pl.ds(g * L, L)
            iwu_v[s] = lax.shift_right_logical(idxu_v[s], 3)
            iwi_v[s] = lax.shift_right_logical(idxi_v[s], 3)

        gubufs = (gu0, gu1)
        gibufs = (gi0, gi1)

        def fire(c):
            s = pl.ds(c * CHUNK, CHUNK)
            return (
                pltpu.async_copy(uemb_hbm.at[iwu_v.at[s]], gubufs[c % 2], sem_g),
                pltpu.async_copy(iemb_hbm.at[iwi_v.at[s]], gibufs[c % 2], sem_g),
            )

        wf = [wmat_v[f] for f in range(L)]
        breg = bv_v[...]
        lanes = lax.iota(jnp.int32, L)

        pend = fire(0)
        for c in range(NCHUNK):
            nxt = fire(c + 1) if c + 1 < NCHUNK else None
            for cp_ in pend:
                cp_.wait()
            pend = nxt
            gu = gubufs[c % 2]
            gi = gibufs[c % 2]

            @pl.loop(0, CHUNK // L)
            def _(g):
                s = pl.ds(c * CHUNK + g * L, L)
                rvec = g * L + lanes
                au = lax.shift_left(idxu_v[s] & 7, 4)
                ai = lax.shift_left(idxi_v[s] & 7, 4)
                acc = breg
                for f in range(L):
                    xu = plsc.load_gather(gu, [rvec, au + f])
                    xi = plsc.load_gather(gi, [rvec, ai + f])
                    acc = acc + xu * xi * wf[f]
                out_v[s] = acc

        pltpu.sync_copy(out_v, out_hbm.at[pl.ds(base, BPW)])

    return sc_kernel(user, item, uw, iw, wmat, bv)
